# baseline (speedup 1.0000x reference)
import functools

import jax
import jax.numpy as jnp
from jax import lax
from jax.experimental import pallas as pl
from jax.experimental.pallas import tpu as pltpu
from jax.experimental.pallas import tpu_sc as plsc

VOCAB = 1000
VPAD = 1024
LPR = VPAD // 128
NC = 2
NS = 16
NW = NC * NS


def _make_gather(bt, ktok):
    kr = ktok * LPR
    b_per_w = bt // NW
    nchunk = b_per_w // ktok
    assert nchunk % 2 == 0 and nchunk >= 4 and kr <= 128
    mesh = plsc.VectorSubcoreMesh(core_axis_name="c", subcore_axis_name="s")

    @functools.partial(
        pl.kernel,
        out_type=jax.ShapeDtypeStruct((bt * LPR, 128), jnp.float32),
        mesh=mesh,
        scratch_types=[
            pltpu.VMEM((b_per_w * LPR,), jnp.int32),
            pltpu.VMEM((2, kr, 128), jnp.float32),
            pltpu.SemaphoreType.DMA,
            pltpu.SemaphoreType.DMA,
        ],
        compiler_params=pltpu.CompilerParams(use_tc_tiling_on_sc=False),
    )
    def gather_kernel(tview_hbm, idx_hbm, out_hbm, idx_v, rows_v, sem0, sem1):
        wid = lax.axis_index("s") * NC + lax.axis_index("c")
        base = wid * b_per_w * LPR
        sems = (sem0, sem1)
        pltpu.sync_copy(idx_hbm.at[pl.ds(base, b_per_w * LPR)], idx_v)

        def gather_dma(c, slot):
            return pltpu.make_async_copy(
                tview_hbm.at[idx_v.at[pl.ds(c * kr, kr)]],
                rows_v.at[slot],
                sems[slot],
            )

        def out_copy(c, slot):
            pltpu.sync_copy(rows_v.at[slot], out_hbm.at[pl.ds(base + c * kr, kr)])

        gather_dma(0, 0).start()

        def body(c2, carry):
            c = 2 * c2
            gather_dma(c + 1, 1).start()
            gather_dma(c, 0).wait()
            out_copy(c, 0)
            gather_dma(c + 2, 0).start()
            gather_dma(c + 1, 1).wait()
            out_copy(c + 1, 1)
            return carry

        lax.fori_loop(0, nchunk // 2 - 1, body, 0)
        c = nchunk - 2
        gather_dma(c + 1, 1).start()
        gather_dma(c, 0).wait()
        out_copy(c, 0)
        gather_dma(c + 1, 1).wait()
        out_copy(c + 1, 1)

    return gather_kernel


_gather = _make_gather(81920, 16)


@jax.jit
def kernel(idx, token_embedding_table):
    b, t = idx.shape
    flat = idx.reshape(b * t)
    idx8 = (flat[:, None] * LPR + jnp.arange(LPR, dtype=jnp.int32)).reshape(-1)
    table_p = jnp.pad(token_embedding_table, ((0, 0), (0, VPAD - VOCAB)))
    tview = table_p.reshape(VOCAB * LPR, 128)
    out = _gather(tview, idx8)
    return out.reshape(b, t, VPAD)[:, :, :VOCAB]
